# baseline (speedup 1.0000x reference)
import functools

import jax
import jax.numpy as jnp
from jax import lax
from jax.experimental import pallas as pl
from jax.experimental.pallas import tpu as pltpu
from jax.experimental.pallas import tpu_sc as plsc

_TS = 26
_TOT = 126
_REP = 20
_F = _TS + (_TOT - _TS) * _REP
_D = 64
_B = 1024
_HB = _B // 2
_FB = 48
_PAIRS = _FB // 2
_NPAIR = _F // 2
_LANE = 128
_NW = 32
_VL = 16


def _sc_splat_body(tbl_hbm, out_hbm, row_v, buf_v, sem):
    c = lax.axis_index("c")
    s = lax.axis_index("s")
    wid = s * 2 + c

    def do_row(r):
        pltpu.async_copy(tbl_hbm.at[r], row_v, sem).wait()
        for c in range(_D // _VL):
            chunk = row_v[pl.ds(c * _VL, _VL)]
            for e in range(_VL):
                idx = jnp.full((_VL,), e, jnp.int32)
                val = jnp.take(chunk, idx, mode="wrap")
                for m in range(_LANE // _VL):
                    base = (c * _VL + e) * _LANE + m * _VL
                    buf_v[pl.ds(base, _VL)] = val
        pltpu.async_copy(buf_v, out_hbm.at[r], sem).wait()

    def body(k, carry):
        r = wid + _NW * k

        @pl.when(r < _TOT)
        def _():
            do_row(r)

        return carry

    lax.fori_loop(0, (_TOT + _NW - 1) // _NW, body, 0)


def _sc_splat(table):
    mesh = plsc.VectorSubcoreMesh(core_axis_name="c", subcore_axis_name="s")
    f = functools.partial(
        pl.kernel,
        mesh=mesh,
        out_type=jax.ShapeDtypeStruct((_TOT, _D * _LANE), jnp.float32),
        scratch_types=[
            pltpu.VMEM((_D,), jnp.float32),
            pltpu.VMEM((_D * _LANE,), jnp.float32),
            pltpu.SemaphoreType.DMA,
        ],
    )(_sc_splat_body)
    return f(table).reshape(_TOT, _D, _LANE)


def _tc_body(xlo_ref, xhi_ref, spl_ref, o_ref):
    i = pl.program_id(0)
    for j in range(_PAIRS):
        p = jnp.minimum(i * _PAIRS + j, _NPAIR - 1)
        rh = jnp.where(p < _TS // 2, p, _TS // 2 + (p - _TS // 2) % 50)
        pair = spl_ref[pl.ds(2 * rh, 2)]
        bias = jnp.concatenate([pair] * (_HB // _LANE), axis=2)
        o_ref[2 * j:2 * j + 2, :, 0:_HB] = xlo_ref[2 * j:2 * j + 2] + bias
        o_ref[2 * j:2 * j + 2, :, _HB:_B] = xhi_ref[2 * j:2 * j + 2] + bias


def kernel(X, table):
    x_t = jnp.transpose(X, (1, 2, 0))
    spl = _sc_splat(table)

    out = pl.pallas_call(
        _tc_body,
        grid=(pl.cdiv(_F, _FB),),
        in_specs=[
            pl.BlockSpec((_FB, _D, _HB), lambda i: (i, 0, 0)),
            pl.BlockSpec((_FB, _D, _HB), lambda i: (i, 0, 1)),
            pl.BlockSpec((_TOT, _D, _LANE), lambda i: (0, 0, 0)),
        ],
        out_specs=pl.BlockSpec((_FB, _D, _B), lambda i: (i, 0, 0)),
        out_shape=jax.ShapeDtypeStruct((_F, _D, _B), X.dtype),
    )(x_t, x_t, spl)
    return jnp.transpose(out, (2, 0, 1))


# speedup vs baseline: 1.0905x; 1.0905x over previous
import jax
import jax.numpy as jnp
from jax.experimental import pallas as pl
from jax.experimental.pallas import tpu as pltpu

_TS = 26
_TOT = 126
_REP = 20
_F = _TS + (_TOT - _TS) * _REP
_D = 64
_B = 1024
_HB = _B // 2
_FB = 52
_PAIRS = _FB // 2
_NPAIR = _F // 2
_LANE = 128


def _body(xlo_ref, xhi_ref, tbl_ref, o_ref, spl_ref):
    @pl.when(pl.program_id(0) == 0)
    def _init():
        t = tbl_ref[...]
        spl_ref[...] = jnp.broadcast_to(t[:, :, None], (_TOT, _D, _LANE))

    i = pl.program_id(0)
    for j in range(_PAIRS):
        p = jnp.minimum(i * _PAIRS + j, _NPAIR - 1)
        rh = jnp.where(p < _TS // 2, p, _TS // 2 + (p - _TS // 2) % 50)
        pair = spl_ref[pl.ds(2 * rh, 2)]
        bias = jnp.concatenate([pair] * (_HB // _LANE), axis=2)
        o_ref[2 * j:2 * j + 2, :, 0:_HB] = xlo_ref[2 * j:2 * j + 2] + bias
        o_ref[2 * j:2 * j + 2, :, _HB:_B] = xhi_ref[2 * j:2 * j + 2] + bias


def kernel(X, table):
    x_t = jnp.transpose(X, (1, 2, 0))

    out = pl.pallas_call(
        _body,
        grid=(pl.cdiv(_F, _FB),),
        in_specs=[
            pl.BlockSpec((_FB, _D, _HB), lambda i: (i, 0, 0)),
            pl.BlockSpec((_FB, _D, _HB), lambda i: (i, 0, 1)),
            pl.BlockSpec((_TOT, _D), lambda i: (0, 0)),
        ],
        out_specs=pl.BlockSpec((_FB, _D, _B), lambda i: (i, 0, 0)),
        out_shape=jax.ShapeDtypeStruct((_F, _D, _B), X.dtype),
        scratch_shapes=[pltpu.VMEM((_TOT, _D, _LANE), jnp.float32)],
    )(x_t, x_t, table)
    return jnp.transpose(out, (2, 0, 1))


# speedup vs baseline: 1.0907x; 1.0002x over previous
import jax
import jax.numpy as jnp
from jax.experimental import pallas as pl
from jax.experimental.pallas import tpu as pltpu

_TS = 26
_TOT = 126
_REP = 20
_F = _TS + (_TOT - _TS) * _REP
_D = 64
_B = 1024
_HB = _B // 2
_FB = 54
_PAIRS = _FB // 2
_NPAIR = _F // 2
_LANE = 128


def _body(xlo_ref, xhi_ref, tbl_ref, o_ref, spl_ref):
    @pl.when(pl.program_id(0) == 0)
    def _init():
        t = tbl_ref[...]
        spl_ref[...] = jnp.broadcast_to(t[:, :, None], (_TOT, _D, _LANE))

    i = pl.program_id(0)
    for j in range(_PAIRS):
        p = jnp.minimum(i * _PAIRS + j, _NPAIR - 1)
        rh = jnp.where(p < _TS // 2, p, _TS // 2 + (p - _TS // 2) % 50)
        pair = spl_ref[pl.ds(2 * rh, 2)]
        bias = jnp.concatenate([pair] * (_HB // _LANE), axis=2)
        o_ref[2 * j:2 * j + 2, :, 0:_HB] = xlo_ref[2 * j:2 * j + 2] + bias
        o_ref[2 * j:2 * j + 2, :, _HB:_B] = xhi_ref[2 * j:2 * j + 2] + bias


def kernel(X, table):
    x_t = jnp.transpose(X, (1, 2, 0))

    out = pl.pallas_call(
        _body,
        grid=(pl.cdiv(_F, _FB),),
        in_specs=[
            pl.BlockSpec((_FB, _D, _HB), lambda i: (i, 0, 0)),
            pl.BlockSpec((_FB, _D, _HB), lambda i: (i, 0, 1)),
            pl.BlockSpec((_TOT, _D), lambda i: (0, 0)),
        ],
        out_specs=pl.BlockSpec((_FB, _D, _B), lambda i: (i, 0, 0)),
        out_shape=jax.ShapeDtypeStruct((_F, _D, _B), X.dtype),
        scratch_shapes=[pltpu.VMEM((_TOT, _D, _LANE), jnp.float32)],
    )(x_t, x_t, table)
    return jnp.transpose(out, (2, 0, 1))
